# Initial kernel scaffold; baseline (speedup 1.0000x reference)
import jax
import jax.numpy as jnp
from jax.experimental import pallas as pl


def kernel(position_ids, pos_emb_weight):
    raise NotImplementedError("write your pallas kernel here")


# speedup vs baseline: 2.0147x; 2.0147x over previous
import functools

import jax
import jax.numpy as jnp
from jax import lax
from jax.experimental import pallas as pl
from jax.experimental.pallas import tpu as pltpu
from jax.experimental.pallas import tpu_sc as plsc

NC, NS = 2, 16
NW = NC * NS
BATCH, SEQ = 4, 4096
N = BATCH * SEQ
D = 1024
PER_W = N // NW
CHUNK = 32
NCHUNK = PER_W // CHUNK

_mesh = plsc.VectorSubcoreMesh(
    core_axis_name="c", subcore_axis_name="s", num_cores=NC, num_subcores=NS
)


@functools.partial(
    pl.kernel,
    out_type=jax.ShapeDtypeStruct((N, D), jnp.float32),
    mesh=_mesh,
    scratch_types=[
        pltpu.VMEM((NCHUNK, CHUNK), jnp.int32),
        pltpu.VMEM((CHUNK, D), jnp.float32),
        pltpu.VMEM((CHUNK, D), jnp.float32),
        pltpu.SemaphoreType.DMA,
        pltpu.SemaphoreType.DMA,
    ],
)
def _emb_lookup(idx_hbm, table_hbm, out_hbm, idx_v, rows0, rows1, gsem, ssem):
    wid = lax.axis_index("s") * NC + lax.axis_index("c")
    base = wid * PER_W
    pltpu.sync_copy(idx_hbm.at[wid], idx_v)

    bufs = (rows0, rows1)
    gathers = [None] * NCHUNK
    scatters = [None] * NCHUNK
    gathers[0] = pltpu.async_copy(table_hbm.at[idx_v.at[0]], rows0, gsem)
    for j in range(NCHUNK):
        cur = bufs[j % 2]
        if j + 1 < NCHUNK:
            if j >= 1:
                scatters[j - 1].wait()
            gathers[j + 1] = pltpu.async_copy(
                table_hbm.at[idx_v.at[j + 1]], bufs[(j + 1) % 2], gsem
            )
        gathers[j].wait()
        scatters[j] = pltpu.async_copy(
            cur, out_hbm.at[pl.ds(base + j * CHUNK, CHUNK)], ssem
        )
    scatters[NCHUNK - 2].wait()
    scatters[NCHUNK - 1].wait()


def kernel(position_ids, pos_emb_weight):
    ids = position_ids.astype(jnp.int32).reshape(NW, NCHUNK, CHUNK)
    out = _emb_lookup(ids, pos_emb_weight)
    return out.reshape(BATCH, SEQ, D)
